# Initial kernel scaffold; baseline (speedup 1.0000x reference)
import jax
import jax.numpy as jnp
from jax.experimental import pallas as pl


def kernel(x, gate_w, w1, w2, w3):
    raise NotImplementedError("write your pallas kernel here")


# speedup vs baseline: 1.1435x; 1.1435x over previous
import functools

import jax
import jax.numpy as jnp
from jax.experimental import pallas as pl
from jax.experimental.pallas import tpu as pltpu

E = 16
TOPK = 2
H = 1024
FF = 3584
T = 64

BF = 896
NF = FF // BF


def _moe_body(x_ref, gate_w_ref, w1_ref, w2_ref, w3_ref, out_ref, cw_ref):
    e = pl.program_id(0)
    f = pl.program_id(1)

    @pl.when((e == 0) & (f == 0))
    def _routing():
        x = x_ref[...]
        logits = jax.lax.dot_general(
            x, gate_w_ref[...], (((1,), (1,)), ((), ())),
            preferred_element_type=jnp.float32)
        p = jax.nn.softmax(logits, axis=-1)
        idx = jax.lax.broadcasted_iota(jnp.int32, (T, E), 1)
        m1 = jnp.max(p, axis=-1, keepdims=True)
        i1 = jnp.min(jnp.where(p == m1, idx, E), axis=-1, keepdims=True)
        mask1 = idx == i1
        p2 = jnp.where(mask1, -1.0, p)
        m2 = jnp.max(p2, axis=-1, keepdims=True)
        i2 = jnp.min(jnp.where(p2 == m2, idx, E), axis=-1, keepdims=True)
        mask2 = idx == i2
        s = m1 + m2
        cw = (jnp.where(mask1, m1, 0.0) + jnp.where(mask2, m2, 0.0)) / s
        cw_ref[:, 0:E] = cw
        out_ref[...] = jnp.zeros_like(out_ref)

    x = x_ref[...]
    w1b = w1_ref[0]
    w3b = w3_ref[0]
    w2b = w2_ref[0]
    gate = jax.lax.dot_general(
        x, w1b, (((1,), (1,)), ((), ())), preferred_element_type=jnp.float32)
    up = jax.lax.dot_general(
        x, w3b, (((1,), (1,)), ((), ())), preferred_element_type=jnp.float32)
    inter = gate * jax.lax.logistic(gate) * up
    partial = jax.lax.dot_general(
        inter, w2b, (((1,), (1,)), ((), ())), preferred_element_type=jnp.float32)
    lanes = jax.lax.broadcasted_iota(jnp.int32, (T, 128), 1)
    cw_col = jnp.sum(jnp.where(lanes == e, cw_ref[...], 0.0),
                     axis=-1, keepdims=True)
    out_ref[...] += cw_col * partial


@jax.jit
def kernel(x, gate_w, w1, w2, w3):
    return pl.pallas_call(
        _moe_body,
        grid=(E, NF),
        in_specs=[
            pl.BlockSpec((T, H), lambda e, f: (0, 0)),
            pl.BlockSpec((E, H), lambda e, f: (0, 0)),
            pl.BlockSpec((1, BF, H), lambda e, f: (e, f, 0)),
            pl.BlockSpec((1, H, BF), lambda e, f: (e, 0, f)),
            pl.BlockSpec((1, BF, H), lambda e, f: (e, f, 0)),
        ],
        out_specs=pl.BlockSpec((T, H), lambda e, f: (0, 0)),
        out_shape=jax.ShapeDtypeStruct((T, H), jnp.float32),
        scratch_shapes=[pltpu.VMEM((T, 128), jnp.float32)],
        compiler_params=pltpu.CompilerParams(
            dimension_semantics=("arbitrary", "arbitrary"),
        ),
    )(x, gate_w, w1, w2, w3)
